# Initial kernel scaffold; baseline (speedup 1.0000x reference)
import jax
import jax.numpy as jnp
from jax.experimental import pallas as pl


def kernel(x, edata, edge_index, We, be, Wn, bn):
    raise NotImplementedError("write your pallas kernel here")


# speedup vs baseline: 5.1339x; 5.1339x over previous
import functools

import jax
import jax.numpy as jnp
from jax import lax
from jax.experimental import pallas as pl
from jax.experimental.pallas import tpu as pltpu
from jax.experimental.pallas import tpu_sc as plsc

N_NODES = 10000
N_EDGES = 320000
D_FEAT = 128
D_EDGE = 16
E_OUT = 16
N_OUT = 128

NC = 2
NS = 16
CHUNK = 512
G = 128
NG = CHUNK // G
NSTEPS = N_EDGES // CHUNK
N_PAD = 10240
ROWS_PER_SUB = N_PAD // NS

F32 = jnp.float32


def _nodeproj_body(x_ref, wsT_ref, wdT_ref, xs_ref, xd_ref):
    xb = x_ref[...]
    xs_ref[...] = jnp.dot(xb, wsT_ref[...], preferred_element_type=F32)
    xd_ref[...] = jnp.dot(xb, wdT_ref[...], preferred_element_type=F32)


def _node_proj(x, wsT, wdT):
    blk = 2048
    return pl.pallas_call(
        _nodeproj_body,
        grid=(N_PAD // blk,),
        in_specs=[
            pl.BlockSpec((blk, D_FEAT), lambda i: (i, 0)),
            pl.BlockSpec((D_FEAT, E_OUT), lambda i: (0, 0)),
            pl.BlockSpec((D_FEAT, E_OUT), lambda i: (0, 0)),
        ],
        out_specs=[
            pl.BlockSpec((blk, E_OUT), lambda i: (i, 0)),
            pl.BlockSpec((blk, E_OUT), lambda i: (i, 0)),
        ],
        out_shape=[
            jax.ShapeDtypeStruct((N_PAD, E_OUT), F32),
            jax.ShapeDtypeStruct((N_PAD, E_OUT), F32),
        ],
    )(x, wsT, wdT)


def _edgeproj_body(e_ref, weT_ref, be_ref, o_ref):
    o_ref[...] = (
        jnp.dot(e_ref[...], weT_ref[...], preferred_element_type=F32)
        + be_ref[...]
    )


def _edge_proj(edata, weT, be2):
    blk = 4000
    return pl.pallas_call(
        _edgeproj_body,
        grid=(N_EDGES // blk,),
        in_specs=[
            pl.BlockSpec((blk, D_EDGE), lambda i: (i, 0)),
            pl.BlockSpec((D_EDGE, E_OUT), lambda i: (0, 0)),
            pl.BlockSpec((1, E_OUT), lambda i: (0, 0)),
        ],
        out_specs=pl.BlockSpec((blk, E_OUT), lambda i: (i, 0)),
        out_shape=jax.ShapeDtypeStruct((N_EDGES, E_OUT), F32),
    )(edata, weT, be2)


def _final_body(x_ref, sp_ref, cp_ref, wxT_ref, wnT_ref, bn_ref, o_ref):
    sums = sp_ref[0] + sp_ref[1]
    cnt = cp_ref[0, :, 0:1] + cp_ref[1, :, 0:1]
    ndata = sums / jnp.maximum(cnt, 1.0)
    o_ref[...] = (
        jnp.dot(x_ref[...], wxT_ref[...], preferred_element_type=F32)
        + jnp.dot(ndata, wnT_ref[...], preferred_element_type=F32)
        + bn_ref[...]
    )


def _final_proj(x, sums_p, cnts_p, wxT, wnT, bn2):
    blk = 2000
    return pl.pallas_call(
        _final_body,
        grid=(N_NODES // blk,),
        in_specs=[
            pl.BlockSpec((blk, D_FEAT), lambda i: (i, 0)),
            pl.BlockSpec((NC, blk, E_OUT), lambda i: (0, i, 0)),
            pl.BlockSpec((NC, blk, E_OUT), lambda i: (0, i, 0)),
            pl.BlockSpec((D_FEAT, N_OUT), lambda i: (0, 0)),
            pl.BlockSpec((E_OUT, N_OUT), lambda i: (0, 0)),
            pl.BlockSpec((1, N_OUT), lambda i: (0, 0)),
        ],
        out_specs=pl.BlockSpec((blk, N_OUT), lambda i: (i, 0)),
        out_shape=jax.ShapeDtypeStruct((N_NODES, N_OUT), F32),
    )(x, sums_p, cnts_p, wxT, wnT, bn2)


def _sc_edge_body(eproj_hbm, row_hbm, col_hbm, xs_hbm, xd_hbm,
                  enew_hbm, sums_hbm, cnts_hbm,
                  acc_s, acc_c, xs_sh, xd_sh, xs_c, xd_c, ones_c, zbuf):
    c = lax.axis_index("core")
    s = lax.axis_index("subcore")

    @pl.loop(0, ROWS_PER_SUB)
    def _(i):
        zbuf[i, :] = jnp.zeros((E_OUT,), F32)

    @pl.loop(0, G)
    def _(i):
        ones_c[i, :] = jnp.ones((E_OUT,), F32)

    base = s * ROWS_PER_SUB
    pltpu.sync_copy(zbuf, acc_s.at[pl.ds(base, ROWS_PER_SUB)])
    pltpu.sync_copy(zbuf, acc_c.at[pl.ds(base, ROWS_PER_SUB)])
    pltpu.sync_copy(xs_hbm.at[pl.ds(base, ROWS_PER_SUB)],
                    xs_sh.at[pl.ds(base, ROWS_PER_SUB)])
    pltpu.sync_copy(xd_hbm.at[pl.ds(base, ROWS_PER_SUB)],
                    xd_sh.at[pl.ds(base, ROWS_PER_SUB)])
    plsc.subcore_barrier()

    def body(eproj_v, row_v, col_v, out_v):
        for g in range(NG):
            pltpu.sync_copy(xs_sh.at[row_v.at[0, g]],
                            xs_c.at[pl.ds(g * G, G)])
            pltpu.sync_copy(xd_sh.at[col_v.at[0, g]],
                            xd_c.at[pl.ds(g * G, G)])

        @pl.loop(0, CHUNK)
        def _(e):
            out_v[e, :] = eproj_v[e, :] + xs_c[e, :] + xd_c[e, :]

        for g in range(NG):
            pltpu.sync_copy(out_v.at[pl.ds(g * G, G)],
                            acc_s.at[row_v.at[0, g]], add=True)
            pltpu.sync_copy(ones_c, acc_c.at[row_v.at[0, g]], add=True)

    pltpu.emit_pipeline(
        body,
        grid=(NSTEPS,),
        in_specs=[
            pl.BlockSpec((CHUNK, E_OUT), lambda i: (i, 0)),
            pl.BlockSpec((1, NG, G), lambda i: (i, 0, 0)),
            pl.BlockSpec((1, NG, G), lambda i: (i, 0, 0)),
        ],
        out_specs=[pl.BlockSpec((CHUNK, E_OUT), lambda i: (i, 0))],
        core_axis_name=("core", "subcore"),
        dimension_semantics=(pltpu.PARALLEL,),
    )(eproj_hbm, row_hbm, col_hbm, enew_hbm)

    plsc.subcore_barrier()
    pltpu.sync_copy(acc_s.at[pl.ds(base, ROWS_PER_SUB)],
                    sums_hbm.at[c, pl.ds(base, ROWS_PER_SUB)])
    pltpu.sync_copy(acc_c.at[pl.ds(base, ROWS_PER_SUB)],
                    cnts_hbm.at[c, pl.ds(base, ROWS_PER_SUB)])


def _sc_edge(eproj, row3, col3, xs, xd):
    mesh = plsc.VectorSubcoreMesh(
        core_axis_name="core", subcore_axis_name="subcore")
    f = pl.kernel(
        _sc_edge_body,
        out_type=(
            jax.ShapeDtypeStruct((N_EDGES, E_OUT), F32),
            jax.ShapeDtypeStruct((NC, N_PAD, E_OUT), F32),
            jax.ShapeDtypeStruct((NC, N_PAD, E_OUT), F32),
        ),
        mesh=mesh,
        scratch_types=[
            pltpu.VMEM_SHARED((N_PAD, E_OUT), F32),
            pltpu.VMEM_SHARED((N_PAD, E_OUT), F32),
            pltpu.VMEM_SHARED((N_PAD, E_OUT), F32),
            pltpu.VMEM_SHARED((N_PAD, E_OUT), F32),
            pltpu.VMEM((CHUNK, E_OUT), F32),
            pltpu.VMEM((CHUNK, E_OUT), F32),
            pltpu.VMEM((G, E_OUT), F32),
            pltpu.VMEM((ROWS_PER_SUB, E_OUT), F32),
        ],
        compiler_params=pltpu.CompilerParams(use_tc_tiling_on_sc=False),
    )
    return f(eproj, row3, col3, xs, xd)


def kernel(x, edata, edge_index, We, be, Wn, bn):
    row = edge_index[0].astype(jnp.int32)
    col = edge_index[1].astype(jnp.int32)
    row3 = row.reshape(NSTEPS, NG, G)
    col3 = col.reshape(NSTEPS, NG, G)

    WeT = We.T
    weT = WeT[:D_EDGE]
    wsT = WeT[D_EDGE:D_EDGE + D_FEAT]
    wdT = WeT[D_EDGE + D_FEAT:]
    WnT = Wn.T
    wxT = WnT[:D_FEAT]
    wnT = WnT[D_FEAT:]

    xs, xd = _node_proj(x, wsT, wdT)
    eproj = _edge_proj(edata, weT, be.reshape(1, E_OUT))
    enew, sums_p, cnts_p = _sc_edge(eproj, row3, col3, xs, xd)
    x_new = _final_proj(x, sums_p, cnts_p, wxT, wnT, bn.reshape(1, N_OUT))
    return (x_new, enew)


# speedup vs baseline: 7.4798x; 1.4570x over previous
import functools

import jax
import jax.numpy as jnp
from jax import lax
from jax.experimental import pallas as pl
from jax.experimental.pallas import tpu as pltpu
from jax.experimental.pallas import tpu_sc as plsc

N_NODES = 10000
N_EDGES = 320000
D_FEAT = 128
D_EDGE = 16
E_OUT = 16
N_OUT = 128

NC = 2
NS = 16
CHUNK = 512
G = 128
NG = CHUNK // G
NSTEPS = N_EDGES // CHUNK
N_PAD = 10240
ROWS_PER_SUB = N_PAD // NS

F32 = jnp.float32


def _nodeproj_body(x_ref, wsT_ref, wdT_ref, xs_ref, xd_ref):
    xb = x_ref[...]
    xs_ref[...] = jnp.dot(xb, wsT_ref[...], preferred_element_type=F32)
    xd_ref[...] = jnp.dot(xb, wdT_ref[...], preferred_element_type=F32)


def _node_proj(x, wsT, wdT):
    blk = 2048
    return pl.pallas_call(
        _nodeproj_body,
        grid=(N_PAD // blk,),
        in_specs=[
            pl.BlockSpec((blk, D_FEAT), lambda i: (i, 0)),
            pl.BlockSpec((D_FEAT, E_OUT), lambda i: (0, 0)),
            pl.BlockSpec((D_FEAT, E_OUT), lambda i: (0, 0)),
        ],
        out_specs=[
            pl.BlockSpec((blk, E_OUT), lambda i: (i, 0)),
            pl.BlockSpec((blk, E_OUT), lambda i: (i, 0)),
        ],
        out_shape=[
            jax.ShapeDtypeStruct((N_PAD, E_OUT), F32),
            jax.ShapeDtypeStruct((N_PAD, E_OUT), F32),
        ],
    )(x, wsT, wdT)


EB = 2560
ET = EB // 128


def _edgeproj_body(eT_ref, we_ref, be_ref, o_ref):
    t = jnp.dot(we_ref[...], eT_ref[...],
                preferred_element_type=F32) + be_ref[...]
    o_ref[...] = t.reshape(2, 8, ET, 128).transpose(0, 2, 1, 3)


def _edge_proj(edataT, we_e, bec):
    return pl.pallas_call(
        _edgeproj_body,
        grid=(N_EDGES // EB,),
        in_specs=[
            pl.BlockSpec((D_EDGE, EB), lambda i: (0, i)),
            pl.BlockSpec((E_OUT, D_EDGE), lambda i: (0, 0)),
            pl.BlockSpec((E_OUT, 1), lambda i: (0, 0)),
        ],
        out_specs=pl.BlockSpec((2, ET, 8, 128), lambda i: (0, i, 0, 0)),
        out_shape=jax.ShapeDtypeStruct((2, N_EDGES // 128, 8, 128), F32),
    )(edataT, we_e, bec)


def _final_body(x_ref, sp_ref, cp_ref, wxT_ref, wnT_ref, bn_ref, o_ref):
    sp = sp_ref[...][:, :N_NODES]
    cp = cp_ref[...][:, :N_NODES]
    sums = sp[0] + sp[1]
    cnt = cp[0, :, 0:1] + cp[1, :, 0:1]
    ndata = sums / jnp.maximum(cnt, 1.0)
    o_ref[...] = (
        jnp.dot(x_ref[...], wxT_ref[...], preferred_element_type=F32)
        + jnp.dot(ndata, wnT_ref[...], preferred_element_type=F32)
        + bn_ref[...]
    )


def _final_proj(x, sums_p, cnts_p, wxT, wnT, bn2):
    return pl.pallas_call(
        _final_body,
        grid=(1,),
        in_specs=[
            pl.BlockSpec((N_NODES, D_FEAT), lambda i: (0, 0)),
            pl.BlockSpec((NC, N_PAD, E_OUT), lambda i: (0, 0, 0)),
            pl.BlockSpec((NC, N_PAD, E_OUT), lambda i: (0, 0, 0)),
            pl.BlockSpec((D_FEAT, N_OUT), lambda i: (0, 0)),
            pl.BlockSpec((E_OUT, N_OUT), lambda i: (0, 0)),
            pl.BlockSpec((1, N_OUT), lambda i: (0, 0)),
        ],
        out_specs=pl.BlockSpec((N_NODES, N_OUT), lambda i: (0, 0)),
        out_shape=jax.ShapeDtypeStruct((N_NODES, N_OUT), F32),
    )(x, sums_p, cnts_p, wxT, wnT, bn2)


def _sc_edge_body(ep_hbm, row_hbm, col_hbm, xs_hbm, xd_hbm,
                  enew_hbm, sums_hbm, cnts_hbm,
                  acc_s, acc_c, xs_sh, xd_sh, xs_c, xd_c, rows_c,
                  ones_c, zbuf):
    c = lax.axis_index("core")
    s = lax.axis_index("subcore")
    band_i = lax.iota(jnp.int32, 16) // 8
    feat_i = lax.iota(jnp.int32, 16) % 8

    @pl.loop(0, ROWS_PER_SUB)
    def _(i):
        zbuf[i, :] = jnp.zeros((E_OUT,), F32)

    @pl.loop(0, G)
    def _(i):
        ones_c[i, :] = jnp.ones((E_OUT,), F32)

    base = s * ROWS_PER_SUB
    pltpu.sync_copy(zbuf, acc_s.at[pl.ds(base, ROWS_PER_SUB)])
    pltpu.sync_copy(zbuf, acc_c.at[pl.ds(base, ROWS_PER_SUB)])
    pltpu.sync_copy(xs_hbm.at[pl.ds(base, ROWS_PER_SUB)],
                    xs_sh.at[pl.ds(base, ROWS_PER_SUB)])
    pltpu.sync_copy(xd_hbm.at[pl.ds(base, ROWS_PER_SUB)],
                    xd_sh.at[pl.ds(base, ROWS_PER_SUB)])
    plsc.subcore_barrier()

    def body(ep_v, row_v, col_v, out_v):
        for g in range(NG):
            pltpu.sync_copy(xs_sh.at[row_v.at[0, g]],
                            xs_c.at[pl.ds(g * G, G)])
            pltpu.sync_copy(xd_sh.at[col_v.at[0, g]],
                            xd_c.at[pl.ds(g * G, G)])

        @pl.loop(0, CHUNK)
        def _(e):
            tv = jnp.full((16,), e // 128, jnp.int32)
            cv = jnp.full((16,), e % 128, jnp.int32)
            idx = [band_i, tv, feat_i, cv]
            v = plsc.load_gather(ep_v, idx)
            v = v + xs_c[e, :] + xd_c[e, :]
            rows_c[e, :] = v
            plsc.store_scatter(out_v, idx, v)

        for g in range(NG):
            pltpu.sync_copy(rows_c.at[pl.ds(g * G, G)],
                            acc_s.at[row_v.at[0, g]], add=True)
            pltpu.sync_copy(ones_c, acc_c.at[row_v.at[0, g]], add=True)

    pltpu.emit_pipeline(
        body,
        grid=(NSTEPS,),
        in_specs=[
            pl.BlockSpec((2, NG, 8, G), lambda i: (0, i, 0, 0)),
            pl.BlockSpec((1, NG, G), lambda i: (i, 0, 0)),
            pl.BlockSpec((1, NG, G), lambda i: (i, 0, 0)),
        ],
        out_specs=[pl.BlockSpec((2, NG, 8, G), lambda i: (0, i, 0, 0))],
        core_axis_name=("core", "subcore"),
        dimension_semantics=(pltpu.PARALLEL,),
    )(ep_hbm, row_hbm, col_hbm, enew_hbm)

    plsc.subcore_barrier()
    pltpu.sync_copy(acc_s.at[pl.ds(base, ROWS_PER_SUB)],
                    sums_hbm.at[c, pl.ds(base, ROWS_PER_SUB)])
    pltpu.sync_copy(acc_c.at[pl.ds(base, ROWS_PER_SUB)],
                    cnts_hbm.at[c, pl.ds(base, ROWS_PER_SUB)])


def _sc_edge(ep4, row3, col3, xs, xd):
    mesh = plsc.VectorSubcoreMesh(
        core_axis_name="core", subcore_axis_name="subcore")
    f = pl.kernel(
        _sc_edge_body,
        out_type=(
            jax.ShapeDtypeStruct((2, N_EDGES // 128, 8, G), F32),
            jax.ShapeDtypeStruct((NC, N_PAD, E_OUT), F32),
            jax.ShapeDtypeStruct((NC, N_PAD, E_OUT), F32),
        ),
        mesh=mesh,
        scratch_types=[
            pltpu.VMEM_SHARED((N_PAD, E_OUT), F32),
            pltpu.VMEM_SHARED((N_PAD, E_OUT), F32),
            pltpu.VMEM_SHARED((N_PAD, E_OUT), F32),
            pltpu.VMEM_SHARED((N_PAD, E_OUT), F32),
            pltpu.VMEM((CHUNK, E_OUT), F32),
            pltpu.VMEM((CHUNK, E_OUT), F32),
            pltpu.VMEM((CHUNK, E_OUT), F32),
            pltpu.VMEM((G, E_OUT), F32),
            pltpu.VMEM((ROWS_PER_SUB, E_OUT), F32),
        ],
        compiler_params=pltpu.CompilerParams(
            use_tc_tiling_on_sc=False, needs_layout_passes=False),
    )
    return f(ep4, row3, col3, xs, xd)


def kernel(x, edata, edge_index, We, be, Wn, bn):
    row = edge_index[0].astype(jnp.int32)
    col = edge_index[1].astype(jnp.int32)
    row3 = row.reshape(NSTEPS, NG, G)
    col3 = col.reshape(NSTEPS, NG, G)

    WeT = We.T
    weT = WeT[:D_EDGE]
    wsT = WeT[D_EDGE:D_EDGE + D_FEAT]
    wdT = WeT[D_EDGE + D_FEAT:]
    WnT = Wn.T
    wxT = WnT[:D_FEAT]
    wnT = WnT[D_FEAT:]

    xs, xd = _node_proj(x, wsT, wdT)
    we_e = We[:, :D_EDGE]
    ep4 = _edge_proj(edata.T, we_e, be.reshape(E_OUT, 1))
    enew4, sums_p, cnts_p = _sc_edge(ep4, row3, col3, xs, xd)
    edata_new = (enew4.transpose(0, 2, 1, 3)
                 .reshape(E_OUT, N_EDGES).T)
    x_new = _final_proj(x, sums_p, cnts_p, wxT, wnT,
                        bn.reshape(1, N_OUT))
    return (x_new, edata_new)


# speedup vs baseline: 7.8181x; 1.0452x over previous
import functools

import jax
import jax.numpy as jnp
from jax import lax
from jax.experimental import pallas as pl
from jax.experimental.pallas import tpu as pltpu
from jax.experimental.pallas import tpu_sc as plsc

N_NODES = 10000
N_EDGES = 320000
D_FEAT = 128
D_EDGE = 16
E_OUT = 16
N_OUT = 128

NC = 2
NS = 16
CHUNK = 512
G = 128
NG = CHUNK // G
NSTEPS = N_EDGES // CHUNK
N_PAD = 10240
ROWS_PER_SUB = N_PAD // NS

F32 = jnp.float32


def _nodeproj_body(x_ref, wsT_ref, wdT_ref, xs_ref, xd_ref):
    xb = x_ref[...]
    xs_ref[...] = jnp.dot(xb, wsT_ref[...], preferred_element_type=F32)
    xd_ref[...] = jnp.dot(xb, wdT_ref[...], preferred_element_type=F32)


def _node_proj(x, wsT, wdT):
    blk = 2048
    return pl.pallas_call(
        _nodeproj_body,
        grid=(N_PAD // blk,),
        in_specs=[
            pl.BlockSpec((blk, D_FEAT), lambda i: (i, 0)),
            pl.BlockSpec((D_FEAT, E_OUT), lambda i: (0, 0)),
            pl.BlockSpec((D_FEAT, E_OUT), lambda i: (0, 0)),
        ],
        out_specs=[
            pl.BlockSpec((blk, E_OUT), lambda i: (i, 0)),
            pl.BlockSpec((blk, E_OUT), lambda i: (i, 0)),
        ],
        out_shape=[
            jax.ShapeDtypeStruct((N_PAD, E_OUT), F32),
            jax.ShapeDtypeStruct((N_PAD, E_OUT), F32),
        ],
    )(x, wsT, wdT)


EB = 2560
ET = EB // 128


def _edgeproj_body(eT_ref, we_ref, be_ref, o_ref):
    o_ref[...] = jnp.dot(we_ref[...], eT_ref[...],
                         preferred_element_type=F32) + be_ref[...]


def _edge_proj(edataT, we_e, bec):
    return pl.pallas_call(
        _edgeproj_body,
        grid=(N_EDGES // EB,),
        in_specs=[
            pl.BlockSpec((D_EDGE, EB), lambda i: (0, i)),
            pl.BlockSpec((E_OUT, D_EDGE), lambda i: (0, 0)),
            pl.BlockSpec((E_OUT, 1), lambda i: (0, 0)),
        ],
        out_specs=pl.BlockSpec((E_OUT, EB), lambda i: (0, i)),
        out_shape=jax.ShapeDtypeStruct((E_OUT, N_EDGES), F32),
    )(edataT, we_e, bec)


def _final_body(x_ref, sp_ref, cp_ref, wxT_ref, wnT_ref, bn_ref, o_ref):
    sp = sp_ref[...][:, :N_NODES]
    cp = cp_ref[...][:, :N_NODES]
    sums = sp[0] + sp[1]
    cnt = cp[0, :, 0:1] + cp[1, :, 0:1]
    ndata = sums / jnp.maximum(cnt, 1.0)
    o_ref[...] = (
        jnp.dot(x_ref[...], wxT_ref[...], preferred_element_type=F32)
        + jnp.dot(ndata, wnT_ref[...], preferred_element_type=F32)
        + bn_ref[...]
    )


def _final_proj(x, sums_p, cnts_p, wxT, wnT, bn2):
    return pl.pallas_call(
        _final_body,
        grid=(1,),
        in_specs=[
            pl.BlockSpec((N_NODES, D_FEAT), lambda i: (0, 0)),
            pl.BlockSpec((NC, N_PAD, E_OUT), lambda i: (0, 0, 0)),
            pl.BlockSpec((NC, N_PAD, E_OUT), lambda i: (0, 0, 0)),
            pl.BlockSpec((D_FEAT, N_OUT), lambda i: (0, 0)),
            pl.BlockSpec((E_OUT, N_OUT), lambda i: (0, 0)),
            pl.BlockSpec((1, N_OUT), lambda i: (0, 0)),
        ],
        out_specs=pl.BlockSpec((N_NODES, N_OUT), lambda i: (0, 0)),
        out_shape=jax.ShapeDtypeStruct((N_NODES, N_OUT), F32),
    )(x, sums_p, cnts_p, wxT, wnT, bn2)


def _sc_edge_body(ep_hbm, row_hbm, col_hbm, xs_hbm, xd_hbm,
                  enew_hbm, sums_hbm, cnts_hbm,
                  acc_s, acc_c, xs_sh, xd_sh, xs_c, xd_c, rows_c,
                  ones_c, zbuf, sem_g, sem_s):
    c = lax.axis_index("core")
    s = lax.axis_index("subcore")
    ii = lax.iota(jnp.int32, 16)
    bfi = (ii // 8) * (NG * 1024) + (ii % 8) * G

    @pl.loop(0, ROWS_PER_SUB)
    def _(i):
        zbuf[i, :] = jnp.zeros((E_OUT,), F32)

    @pl.loop(0, G)
    def _(i):
        ones_c[i, :] = jnp.ones((E_OUT,), F32)

    base = s * ROWS_PER_SUB
    pltpu.sync_copy(zbuf, acc_s.at[pl.ds(base, ROWS_PER_SUB)])
    pltpu.sync_copy(zbuf, acc_c.at[pl.ds(base, ROWS_PER_SUB)])
    pltpu.sync_copy(xs_hbm.at[pl.ds(base, ROWS_PER_SUB)],
                    xs_sh.at[pl.ds(base, ROWS_PER_SUB)])
    pltpu.sync_copy(xd_hbm.at[pl.ds(base, ROWS_PER_SUB)],
                    xd_sh.at[pl.ds(base, ROWS_PER_SUB)])
    plsc.subcore_barrier()

    def body(ep_v, row_v, col_v, out_v):
        ds = []
        for g in range(NG):
            ds.append(pltpu.async_copy(
                xs_sh.at[row_v.at[0, g]], xs_c.at[pl.ds(g * G, G)], sem_g))
            ds.append(pltpu.async_copy(
                xd_sh.at[col_v.at[0, g]], xd_c.at[pl.ds(g * G, G)], sem_g))
        for d in ds:
            d.wait()

        for t in range(NG):
            base = bfi + t * 1024

            @pl.loop(0, G)
            def _(ce):
                e = t * G + ce
                idx = base + ce
                v = plsc.load_gather(ep_v, [idx])
                v = v + xs_c[e, :] + xd_c[e, :]
                rows_c[e, :] = v
                plsc.store_scatter(out_v, [idx], v)

        ds = []
        for g in range(NG):
            ds.append(pltpu.async_copy(
                rows_c.at[pl.ds(g * G, G)], acc_s.at[row_v.at[0, g]],
                sem_s, add=True))
            ds.append(pltpu.async_copy(
                ones_c, acc_c.at[row_v.at[0, g]], sem_s, add=True))
        for d in ds:
            d.wait()

    pltpu.emit_pipeline(
        body,
        grid=(NSTEPS,),
        in_specs=[
            pl.BlockSpec((CHUNK * E_OUT,), lambda i: (i,)),
            pl.BlockSpec((1, NG, G), lambda i: (i, 0, 0)),
            pl.BlockSpec((1, NG, G), lambda i: (i, 0, 0)),
        ],
        out_specs=[pl.BlockSpec((CHUNK * E_OUT,), lambda i: (i,))],
        core_axis_name=("core", "subcore"),
        dimension_semantics=(pltpu.PARALLEL,),
    )(ep_hbm, row_hbm, col_hbm, enew_hbm)

    plsc.subcore_barrier()
    pltpu.sync_copy(acc_s.at[pl.ds(base, ROWS_PER_SUB)],
                    sums_hbm.at[c, pl.ds(base, ROWS_PER_SUB)])
    pltpu.sync_copy(acc_c.at[pl.ds(base, ROWS_PER_SUB)],
                    cnts_hbm.at[c, pl.ds(base, ROWS_PER_SUB)])


def _sc_edge(ep4, row3, col3, xs, xd):
    mesh = plsc.VectorSubcoreMesh(
        core_axis_name="core", subcore_axis_name="subcore")
    f = pl.kernel(
        _sc_edge_body,
        out_type=(
            jax.ShapeDtypeStruct((N_EDGES * E_OUT,), F32),
            jax.ShapeDtypeStruct((NC, N_PAD, E_OUT), F32),
            jax.ShapeDtypeStruct((NC, N_PAD, E_OUT), F32),
        ),
        mesh=mesh,
        scratch_types=[
            pltpu.VMEM_SHARED((N_PAD, E_OUT), F32),
            pltpu.VMEM_SHARED((N_PAD, E_OUT), F32),
            pltpu.VMEM_SHARED((N_PAD, E_OUT), F32),
            pltpu.VMEM_SHARED((N_PAD, E_OUT), F32),
            pltpu.VMEM((CHUNK, E_OUT), F32),
            pltpu.VMEM((CHUNK, E_OUT), F32),
            pltpu.VMEM((CHUNK, E_OUT), F32),
            pltpu.VMEM((G, E_OUT), F32),
            pltpu.VMEM((ROWS_PER_SUB, E_OUT), F32),
            pltpu.SemaphoreType.DMA,
            pltpu.SemaphoreType.DMA,
        ],
        compiler_params=pltpu.CompilerParams(
            use_tc_tiling_on_sc=False, needs_layout_passes=False),
    )
    return f(ep4, row3, col3, xs, xd)


def kernel(x, edata, edge_index, We, be, Wn, bn):
    row = edge_index[0].astype(jnp.int32)
    col = edge_index[1].astype(jnp.int32)
    row3 = row.reshape(NSTEPS, NG, G)
    col3 = col.reshape(NSTEPS, NG, G)

    WeT = We.T
    weT = WeT[:D_EDGE]
    wsT = WeT[D_EDGE:D_EDGE + D_FEAT]
    wdT = WeT[D_EDGE + D_FEAT:]
    WnT = Wn.T
    wxT = WnT[:D_FEAT]
    wnT = WnT[D_FEAT:]

    xs, xd = _node_proj(x, wsT, wdT)
    we_e = We[:, :D_EDGE]
    epT = _edge_proj(edata.T, we_e, be.reshape(E_OUT, 1))
    ep_flat = (epT.reshape(2, 8, N_EDGES // 128, 128)
               .transpose(0, 2, 1, 3).reshape(N_EDGES * E_OUT))
    enew_flat, sums_p, cnts_p = _sc_edge(ep_flat, row3, col3, xs, xd)
    edata_new = (enew_flat.reshape(2, N_EDGES // 128, 8, 128)
                 .transpose(0, 2, 1, 3)
                 .reshape(E_OUT, N_EDGES).T)
    x_new = _final_proj(x, sums_p, cnts_p, wxT, wnT,
                        bn.reshape(1, N_OUT))
    return (x_new, edata_new)


# speedup vs baseline: 7.8302x; 1.0015x over previous
import functools

import jax
import jax.numpy as jnp
from jax import lax
from jax.experimental import pallas as pl
from jax.experimental.pallas import tpu as pltpu
from jax.experimental.pallas import tpu_sc as plsc

N_NODES = 10000
N_EDGES = 320000
D_FEAT = 128
D_EDGE = 16
E_OUT = 16
N_OUT = 128

NC = 2
NS = 16
CHUNK = 512
G = 128
NG = CHUNK // G
NSTEPS = N_EDGES // CHUNK
N_PAD = 10240
ROWS_PER_SUB = N_PAD // NS

F32 = jnp.float32


def _nodeproj_body(x_ref, wsT_ref, wdT_ref, xs_ref, xd_ref):
    xb = x_ref[...]
    xs_ref[...] = jnp.dot(xb, wsT_ref[...], preferred_element_type=F32)
    xd_ref[...] = jnp.dot(xb, wdT_ref[...], preferred_element_type=F32)


def _node_proj(x, wsT, wdT):
    blk = 2048
    return pl.pallas_call(
        _nodeproj_body,
        grid=(N_PAD // blk,),
        in_specs=[
            pl.BlockSpec((blk, D_FEAT), lambda i: (i, 0)),
            pl.BlockSpec((D_FEAT, E_OUT), lambda i: (0, 0)),
            pl.BlockSpec((D_FEAT, E_OUT), lambda i: (0, 0)),
        ],
        out_specs=[
            pl.BlockSpec((blk, E_OUT), lambda i: (i, 0)),
            pl.BlockSpec((blk, E_OUT), lambda i: (i, 0)),
        ],
        out_shape=[
            jax.ShapeDtypeStruct((N_PAD, E_OUT), F32),
            jax.ShapeDtypeStruct((N_PAD, E_OUT), F32),
        ],
    )(x, wsT, wdT)


EB = 2560
ET = EB // 128


def _edgeproj_body(eT_ref, we_ref, be_ref, o_ref):
    o_ref[...] = jnp.dot(we_ref[...], eT_ref[...],
                         preferred_element_type=F32) + be_ref[...]


def _edge_proj(edataT, we_e, bec):
    return pl.pallas_call(
        _edgeproj_body,
        grid=(N_EDGES // EB,),
        in_specs=[
            pl.BlockSpec((D_EDGE, EB), lambda i: (0, i)),
            pl.BlockSpec((E_OUT, D_EDGE), lambda i: (0, 0)),
            pl.BlockSpec((E_OUT, 1), lambda i: (0, 0)),
        ],
        out_specs=pl.BlockSpec((E_OUT, EB), lambda i: (0, i)),
        out_shape=jax.ShapeDtypeStruct((E_OUT, N_EDGES), F32),
    )(edataT, we_e, bec)


def _final_body(x_ref, sp_ref, cp_ref, wxT_ref, wnT_ref, bn_ref, o_ref):
    sp = sp_ref[...][:, :N_NODES]
    cp = cp_ref[...][:, :N_NODES]
    sums = sp[0] + sp[1]
    cnt = cp[0, :, 0:1] + cp[1, :, 0:1]
    ndata = sums / jnp.maximum(cnt, 1.0)
    o_ref[...] = (
        jnp.dot(x_ref[...], wxT_ref[...], preferred_element_type=F32)
        + jnp.dot(ndata, wnT_ref[...], preferred_element_type=F32)
        + bn_ref[...]
    )


def _final_proj(x, sums_p, cnts_p, wxT, wnT, bn2):
    return pl.pallas_call(
        _final_body,
        grid=(1,),
        in_specs=[
            pl.BlockSpec((N_NODES, D_FEAT), lambda i: (0, 0)),
            pl.BlockSpec((NC, N_PAD, E_OUT), lambda i: (0, 0, 0)),
            pl.BlockSpec((NC, N_PAD, E_OUT), lambda i: (0, 0, 0)),
            pl.BlockSpec((D_FEAT, N_OUT), lambda i: (0, 0)),
            pl.BlockSpec((E_OUT, N_OUT), lambda i: (0, 0)),
            pl.BlockSpec((1, N_OUT), lambda i: (0, 0)),
        ],
        out_specs=pl.BlockSpec((N_NODES, N_OUT), lambda i: (0, 0)),
        out_shape=jax.ShapeDtypeStruct((N_NODES, N_OUT), F32),
    )(x, sums_p, cnts_p, wxT, wnT, bn2)


def _sc_edge_body(ep_hbm, row_hbm, col_hbm, xs_hbm, xd_hbm,
                  enew_hbm, sums_hbm, cnts_hbm,
                  acc_s, acc_c, xs_sh, xd_sh, xs_c, xd_c, rows_c,
                  ones_c, zbuf, sem_g, sem_s):
    c = lax.axis_index("core")
    s = lax.axis_index("subcore")
    ii = lax.iota(jnp.int32, 16)
    band_i = ii // 8
    feat_i = ii % 8

    @pl.loop(0, ROWS_PER_SUB)
    def _(i):
        zbuf[i, :] = jnp.zeros((E_OUT,), F32)

    @pl.loop(0, G)
    def _(i):
        ones_c[i, :] = jnp.ones((E_OUT,), F32)

    base = s * ROWS_PER_SUB
    pltpu.sync_copy(zbuf, acc_s.at[pl.ds(base, ROWS_PER_SUB)])
    pltpu.sync_copy(zbuf, acc_c.at[pl.ds(base, ROWS_PER_SUB)])
    pltpu.sync_copy(xs_hbm.at[pl.ds(base, ROWS_PER_SUB)],
                    xs_sh.at[pl.ds(base, ROWS_PER_SUB)])
    pltpu.sync_copy(xd_hbm.at[pl.ds(base, ROWS_PER_SUB)],
                    xd_sh.at[pl.ds(base, ROWS_PER_SUB)])
    plsc.subcore_barrier()

    def body(ep_v, row_v, col_v, out_v):
        ds = []
        for g in range(NG):
            ds.append(pltpu.async_copy(
                xs_sh.at[row_v.at[0, g]], xs_c.at[pl.ds(g * G, G)], sem_g))
            ds.append(pltpu.async_copy(
                xd_sh.at[col_v.at[0, g]], xd_c.at[pl.ds(g * G, G)], sem_g))
        for d in ds:
            d.wait()

        for t in range(NG):
            tv = jnp.full((16,), t, jnp.int32)

            @pl.loop(0, G)
            def _(ce):
                e = t * G + ce
                cv = jnp.full((16,), ce, jnp.int32)
                idx = [band_i, tv, feat_i, cv]
                v = plsc.load_gather(ep_v, idx)
                v = v + xs_c[e, :] + xd_c[e, :]
                rows_c[e, :] = v
                plsc.store_scatter(out_v, idx, v)

        ds = []
        for g in range(NG):
            ds.append(pltpu.async_copy(
                rows_c.at[pl.ds(g * G, G)], acc_s.at[row_v.at[0, g]],
                sem_s, add=True))
            ds.append(pltpu.async_copy(
                ones_c, acc_c.at[row_v.at[0, g]], sem_s, add=True))
        for d in ds:
            d.wait()

    pltpu.emit_pipeline(
        body,
        grid=(NSTEPS,),
        in_specs=[
            pl.BlockSpec((2, NG, 8, G), lambda i: (0, i, 0, 0)),
            pl.BlockSpec((1, NG, G), lambda i: (i, 0, 0)),
            pl.BlockSpec((1, NG, G), lambda i: (i, 0, 0)),
        ],
        out_specs=[pl.BlockSpec((2, NG, 8, G), lambda i: (0, i, 0, 0))],
        core_axis_name=("core", "subcore"),
        dimension_semantics=(pltpu.PARALLEL,),
    )(ep_hbm, row_hbm, col_hbm, enew_hbm)

    plsc.subcore_barrier()
    pltpu.sync_copy(acc_s.at[pl.ds(base, ROWS_PER_SUB)],
                    sums_hbm.at[c, pl.ds(base, ROWS_PER_SUB)])
    pltpu.sync_copy(acc_c.at[pl.ds(base, ROWS_PER_SUB)],
                    cnts_hbm.at[c, pl.ds(base, ROWS_PER_SUB)])


def _sc_edge(ep4, row3, col3, xs, xd):
    mesh = plsc.VectorSubcoreMesh(
        core_axis_name="core", subcore_axis_name="subcore")
    f = pl.kernel(
        _sc_edge_body,
        out_type=(
            jax.ShapeDtypeStruct((2, N_EDGES // G, 8, G), F32),
            jax.ShapeDtypeStruct((NC, N_PAD, E_OUT), F32),
            jax.ShapeDtypeStruct((NC, N_PAD, E_OUT), F32),
        ),
        mesh=mesh,
        scratch_types=[
            pltpu.VMEM_SHARED((N_PAD, E_OUT), F32),
            pltpu.VMEM_SHARED((N_PAD, E_OUT), F32),
            pltpu.VMEM_SHARED((N_PAD, E_OUT), F32),
            pltpu.VMEM_SHARED((N_PAD, E_OUT), F32),
            pltpu.VMEM((CHUNK, E_OUT), F32),
            pltpu.VMEM((CHUNK, E_OUT), F32),
            pltpu.VMEM((CHUNK, E_OUT), F32),
            pltpu.VMEM((G, E_OUT), F32),
            pltpu.VMEM((ROWS_PER_SUB, E_OUT), F32),
            pltpu.SemaphoreType.DMA,
            pltpu.SemaphoreType.DMA,
        ],
        compiler_params=pltpu.CompilerParams(
            use_tc_tiling_on_sc=False, needs_layout_passes=False),
    )
    return f(ep4, row3, col3, xs, xd)


def kernel(x, edata, edge_index, We, be, Wn, bn):
    row = edge_index[0].astype(jnp.int32)
    col = edge_index[1].astype(jnp.int32)
    row3 = row.reshape(NSTEPS, NG, G)
    col3 = col.reshape(NSTEPS, NG, G)

    WeT = We.T
    weT = WeT[:D_EDGE]
    wsT = WeT[D_EDGE:D_EDGE + D_FEAT]
    wdT = WeT[D_EDGE + D_FEAT:]
    WnT = Wn.T
    wxT = WnT[:D_FEAT]
    wnT = WnT[D_FEAT:]

    xs, xd = _node_proj(x, wsT, wdT)
    we_e = We[:, :D_EDGE]
    epT = _edge_proj(edata.T, we_e, be.reshape(E_OUT, 1))
    ep4 = (epT.reshape(2, 8, N_EDGES // 128, 128)
           .transpose(0, 2, 1, 3))
    enew4, sums_p, cnts_p = _sc_edge(ep4, row3, col3, xs, xd)
    edata_new = (enew4.transpose(0, 2, 1, 3)
                 .reshape(E_OUT, N_EDGES).T)
    x_new = _final_proj(x, sums_p, cnts_p, wxT, wnT,
                        bn.reshape(1, N_OUT))
    return (x_new, edata_new)
